# baseline (speedup 1.0000x reference)
import jax
import jax.numpy as jnp
from jax.experimental import pallas as pl
from jax.experimental.pallas import tpu as pltpu

_K = 16
_INF = jnp.inf


def _pc_body(xg_ref, geoc_ref, geot_ref, w_ref, wg_ref, b_ref, *rest):
    out_ref = rest[-1]
    mlp = rest[:-1]
    xg = xg_ref[0]
    geot = geot_ref[0]
    geoc = geoc_ref[0]
    w = w_ref[...]
    b = b_ref[...]
    N = xg.shape[0]
    R = geoc.shape[0]
    d = None
    for dim in range(3):
        diff = geoc[:, dim:dim + 1] - geot[dim:dim + 1, :]
        sq = diff * diff
        d = sq if d is None else d + sq
    iota = jax.lax.broadcasted_iota(jnp.int32, (R, N), 1)

    def step(_, carry):
        d, mx = carry
        m = jnp.min(d, axis=1, keepdims=True)
        idx = jnp.min(jnp.where(d == m, iota, N), axis=1, keepdims=True)
        oh = iota == idx
        nbr = jax.lax.dot_general(
            oh.astype(jnp.float32), xg, (((1,), (0,)), ((), ())),
            preferred_element_type=jnp.float32,
            precision=jax.lax.Precision.HIGHEST)
        cat = jnp.concatenate([nbr[:, 0:8], nbr[:, 8:11] - geoc], axis=1)
        v = jnp.maximum(
            jnp.dot(cat, w, preferred_element_type=jnp.float32) + b, 0.0)
        return jnp.where(oh, _INF, d), jnp.maximum(mx, v)

    mx0 = jnp.full((R, 8), -_INF, dtype=jnp.float32)
    _, h = jax.lax.fori_loop(0, _K, step, (d, mx0))
    if mlp:
        wm1, bm1, wm2, bm2, wm3, bm3 = mlp
        h = jnp.maximum(
            jnp.dot(h, wm1[...], preferred_element_type=jnp.float32)
            + bm1[...], 0.0)
        h = jnp.maximum(
            jnp.dot(h, wm2[...], preferred_element_type=jnp.float32)
            + bm2[...], 0.0)
        h = jnp.maximum(
            jnp.dot(h, wm3[...], preferred_element_type=jnp.float32)
            + bm3[...], 0.0)
    out_ref[0] = h


def _full_spec(shape):
    nd = len(shape)
    return pl.BlockSpec(shape, lambda b, r, _n=nd: (0,) * _n)


def _point_conv_call(xg, geoc, geot, w, wg, b, R, mlp_ws=()):
    B, N, _ = xg.shape
    Nout = geoc.shape[1]
    Fout = 6 if mlp_ws else 8
    grid = (B, Nout // R)
    in_specs = [
        pl.BlockSpec((1, N, 11), lambda b_, r: (b_, 0, 0)),
        pl.BlockSpec((1, R, 3), lambda b_, r: (b_, r, 0)),
        pl.BlockSpec((1, 3, N), lambda b_, r: (b_, 0, 0)),
        _full_spec(w.shape),
        _full_spec(wg.shape),
        _full_spec(b.shape),
    ]
    args = [xg, geoc, geot, w, wg, b]
    for w in mlp_ws:
        in_specs.append(_full_spec(w.shape))
        args.append(w)
    return pl.pallas_call(
        _pc_body,
        grid=grid,
        in_specs=in_specs,
        out_specs=pl.BlockSpec((1, R, Fout), lambda b_, r: (b_, r, 0)),
        out_shape=jax.ShapeDtypeStruct((B, Nout, Fout), jnp.float32),
        compiler_params=pltpu.CompilerParams(
            dimension_semantics=("parallel", "parallel")),
    )(*args)


def kernel(x, geoin, Wpc1, bpc1, Wpc2, bpc2, Wm1, bm1, Wm2, bm2, Wm3, bm3):
    B, Ns, _ = x.shape
    N1, N2 = Ns // 2, Ns // 4
    xg1 = jnp.concatenate([x, geoin], axis=-1)
    geot1 = geoin.transpose(0, 2, 1)
    geoc1 = geoin[:, :N1, :]
    h1 = _point_conv_call(xg1, geoc1, geot1, Wpc1, Wpc1[8:11, :],
                          bpc1[None, :], R=512)
    xg2 = jnp.concatenate([h1, geoc1], axis=-1)
    geot2 = geot1[:, :, :N1]
    geoc2 = geoin[:, :N2, :]
    h = _point_conv_call(
        xg2, geoc2, geot2, Wpc2, Wpc2[8:11, :], bpc2[None, :], R=512,
        mlp_ws=(Wm1, bm1[None, :], Wm2, bm2[None, :], Wm3, bm3[None, :]))
    return (h, geoin[:, :N2, :])


# speedup vs baseline: 4.1731x; 4.1731x over previous
import functools

import jax
import jax.numpy as jnp
from jax import lax
from jax.experimental import pallas as pl
from jax.experimental.pallas import tpu as pltpu
from jax.experimental.pallas import tpu_sc as plsc

_K = 16
_INF = jnp.inf
_PAD = 16


def _sel_body(geoc_ref, geot_ref, out_ref, *, N):
    geoc = geoc_ref[0]
    geot = geot_ref[0]
    R = geoc.shape[0]
    d = None
    for dim in range(3):
        diff = geoc[:, dim:dim + 1] - geot[dim:dim + 1, :]
        sq = diff * diff
        d = sq if d is None else d + sq
    iota = lax.broadcasted_iota(jnp.int32, (R, N), 1)
    idxs = []
    for _ in range(_K):
        m = jnp.min(d, axis=1, keepdims=True)
        idx = jnp.min(jnp.where(d == m, iota, N), axis=1, keepdims=True)
        d = jnp.where(iota == idx, _INF, d)
        idxs.append(idx)
    out_ref[0] = jnp.concatenate(idxs, axis=1) + pl.program_id(0) * N


def _select_knn(geoc, geot, R):
    B, Nout, _ = geoc.shape
    N = geot.shape[2]
    return pl.pallas_call(
        functools.partial(_sel_body, N=N),
        grid=(B, Nout // R),
        in_specs=[
            pl.BlockSpec((1, R, 3), lambda b, r: (b, r, 0)),
            pl.BlockSpec((1, 3, N), lambda b, r: (b, 0, 0)),
        ],
        out_specs=pl.BlockSpec((1, R, _K), lambda b, r: (b, r, 0)),
        out_shape=jax.ShapeDtypeStruct((B, Nout, _K), jnp.int32),
        compiler_params=pltpu.CompilerParams(
            dimension_semantics=("parallel", "parallel")),
    )(geoc, geot)


def _sc_gather(table, idx):
    info = plsc.get_sparse_core_info()
    nw = info.num_cores * info.num_subcores
    btot = idx.shape[0]
    b_per_w = btot // nw
    mesh = plsc.VectorSubcoreMesh(core_axis_name="c", subcore_axis_name="s")

    @functools.partial(
        pl.kernel, mesh=mesh,
        compiler_params=pltpu.CompilerParams(use_tc_tiling_on_sc=False),
        out_type=jax.ShapeDtypeStruct((btot, _PAD), jnp.float32),
        scratch_types=[
            pltpu.VMEM((b_per_w,), jnp.int32),
            pltpu.VMEM((b_per_w, _PAD), jnp.float32),
            pltpu.SemaphoreType.DMA,
        ],
    )
    def k(table_hbm, idx_hbm, out_hbm, idx_v, rows_v, sem):
        wid = lax.axis_index("s") * info.num_cores + lax.axis_index("c")
        base = wid * b_per_w
        pltpu.sync_copy(idx_hbm.at[pl.ds(base, b_per_w)], idx_v)
        pltpu.async_copy(table_hbm.at[idx_v], rows_v, sem).wait()
        pltpu.sync_copy(rows_v, out_hbm.at[pl.ds(base, b_per_w)])

    return k(table, idx)


def _mlp_body(g_ref, geoc_ref, w_ref, b_ref, *rest):
    out_ref = rest[-1]
    mlp = rest[:-1]
    g = g_ref[0]
    geoc = geoc_ref[0]
    w = w_ref[...]
    b = b_ref[...]
    mx = None
    for k in range(_K):
        o = k * _PAD
        feat = g[:, o:o + 8]
        rel = g[:, o + 8:o + 11] - geoc
        cat = jnp.concatenate([feat, rel], axis=1)
        v = jnp.maximum(
            jnp.dot(cat, w, preferred_element_type=jnp.float32) + b, 0.0)
        mx = v if mx is None else jnp.maximum(mx, v)
    h = mx
    if mlp:
        wm1, bm1, wm2, bm2, wm3, bm3 = mlp
        h = jnp.maximum(
            jnp.dot(h, wm1[...], preferred_element_type=jnp.float32)
            + bm1[...], 0.0)
        h = jnp.maximum(
            jnp.dot(h, wm2[...], preferred_element_type=jnp.float32)
            + bm2[...], 0.0)
        h = jnp.maximum(
            jnp.dot(h, wm3[...], preferred_element_type=jnp.float32)
            + bm3[...], 0.0)
    out_ref[0] = h


def _full_spec(shape):
    nd = len(shape)
    return pl.BlockSpec(shape, lambda b, r, _n=nd: (0,) * _n)


def _nbr_mlp(g, geoc, w, b, R, mlp_ws=()):
    B, Nout, _ = geoc.shape
    Fout = 6 if mlp_ws else 8
    in_specs = [
        pl.BlockSpec((1, R, _K * _PAD), lambda b_, r: (b_, r, 0)),
        pl.BlockSpec((1, R, 3), lambda b_, r: (b_, r, 0)),
        _full_spec(w.shape),
        _full_spec(b.shape),
    ]
    args = [g, geoc, w, b]
    for ww in mlp_ws:
        in_specs.append(_full_spec(ww.shape))
        args.append(ww)
    return pl.pallas_call(
        _mlp_body,
        grid=(B, Nout // R),
        in_specs=in_specs,
        out_specs=pl.BlockSpec((1, R, Fout), lambda b_, r: (b_, r, 0)),
        out_shape=jax.ShapeDtypeStruct((B, Nout, Fout), jnp.float32),
        compiler_params=pltpu.CompilerParams(
            dimension_semantics=("parallel", "parallel")),
    )(*args)


def _point_conv(xg, geoc, geot, w, b, mlp_ws=()):
    B, N, _ = xg.shape
    Nout = geoc.shape[1]
    idx = _select_knn(geoc, geot, R=min(512, Nout))
    table = jnp.pad(xg, ((0, 0), (0, 0), (0, _PAD - 11)))
    table = table.reshape(B * N, _PAD)
    g = _sc_gather(table, idx.reshape(-1))
    g = g.reshape(B, Nout, _K * _PAD)
    return _nbr_mlp(g, geoc, w, b, R=min(1024, Nout), mlp_ws=mlp_ws)


def kernel(x, geoin, Wpc1, bpc1, Wpc2, bpc2, Wm1, bm1, Wm2, bm2, Wm3, bm3):
    B, Ns, _ = x.shape
    N1, N2 = Ns // 2, Ns // 4
    geot1 = geoin.transpose(0, 2, 1)
    geoc1 = geoin[:, :N1, :]
    xg1 = jnp.concatenate([x, geoin], axis=-1)
    h1 = _point_conv(xg1, geoc1, geot1, Wpc1, bpc1[None, :])
    xg2 = jnp.concatenate([h1, geoc1], axis=-1)
    geot2 = geot1[:, :, :N1]
    geoc2 = geoin[:, :N2, :]
    h = _point_conv(
        xg2, geoc2, geot2, Wpc2, bpc2[None, :],
        mlp_ws=(Wm1, bm1[None, :], Wm2, bm2[None, :], Wm3, bm3[None, :]))
    return (h, geoin[:, :N2, :])
